# baseline (speedup 1.0000x reference)
import functools

import jax
import jax.numpy as jnp
from jax import lax
from jax.experimental import pallas as pl
from jax.experimental.pallas import tpu as pltpu
from jax.experimental.pallas import tpu_sc as plsc

_NUM_ROWS = 16384 * 50
_DIM = 32
_NW = 32
_PER_W = _NUM_ROWS // _NW
_L = 128
_C = 512
_G = _C // _L
_NCHUNK = _PER_W // _C
_GROWS = _PER_W // _L
_NBR = _DIM // 8

_mesh = plsc.VectorSubcoreMesh(core_axis_name="c", subcore_axis_name="s")


@functools.partial(
    pl.kernel,
    mesh=_mesh,
    out_type=jax.ShapeDtypeStruct((_NUM_ROWS * _DIM,), jnp.float32),
    scratch_types=[
        pltpu.VMEM((_GROWS, _L), jnp.int32),
        pltpu.VMEM((2, _C, _DIM), jnp.float32),
        pltpu.VMEM((2, _G * _DIM * _L), jnp.float32),
        pltpu.SemaphoreType.DMA,
        pltpu.SemaphoreType.DMA,
        pltpu.SemaphoreType.DMA,
        pltpu.SemaphoreType.DMA,
    ],
    compiler_params=pltpu.CompilerParams(
        use_tc_tiling_on_sc=False, needs_layout_passes=False),
)
def _emb_gather(w_hbm, idx_hbm, out_hbm, idx_v, rows_v, tiles_v,
                gsem0, gsem1, ssem0, ssem1):
    gsems = (gsem0, gsem1)
    ssems = (ssem0, ssem1)
    wid = lax.axis_index("s") * 2 + lax.axis_index("c")
    gbase = pl.multiple_of(wid * _GROWS, 8)
    pltpu.sync_copy(idx_hbm.at[pl.ds(gbase, _GROWS)], idx_v)
    lanes = lax.iota(jnp.int32, 16)

    def fire_gathers(k, s):
        for j in range(_G):
            pltpu.async_copy(
                w_hbm.at[idx_v.at[k * _G + j]],
                rows_v.at[s].at[pl.ds(j * _L, _L)],
                gsems[s])

    def drain_rows(s):
        pltpu.make_async_copy(
            w_hbm.at[pl.ds(0, _C)], rows_v.at[s], gsems[s]).wait()

    def drain_tiles(s):
        pltpu.make_async_copy(
            out_hbm.at[pl.ds(0, _G * _DIM * _L)], tiles_v.at[s],
            ssems[s]).wait()

    def transpose_store(k, q, drain_stores):
        drain_rows(q)
        if drain_stores:
            drain_tiles(q)
        tiles = tiles_v.at[q]
        d128 = lanes * _L

        rows = rows_v.at[q]

        @plsc.parallel_loop(0, _C // 16, 1, unroll=8)
        def seg_body(u):
            g = u // (_L // 16)
            t_idx = u * 16 + lanes
            tbase = g * (_DIM * _L - _L) + u * 16
            for d in range(_DIM):
                v = plsc.load_gather(
                    rows, [t_idx, jnp.full((16,), d, jnp.int32)])
                tiles[pl.ds(tbase + d * _L, 16)] = v

        for g in range(_G):
            gg = wid * (_NCHUNK * _G) + k * _G + g
            p = gg // (16384 // _L)
            tc = lax.rem(gg, 16384 // _L)
            for br in range(_NBR):
                off = pl.multiple_of(
                    (p * _NBR * (16384 // _L) + br * (16384 // _L) + tc)
                    * (8 * _L), 8)
                pltpu.async_copy(
                    tiles.at[pl.ds(g * (_DIM * _L) + br * (8 * _L), 8 * _L)],
                    out_hbm.at[pl.ds(off, 8 * _L)],
                    ssems[q])

    fire_gathers(0, 0)
    fire_gathers(1, 1)
    transpose_store(0, 0, False)
    fire_gathers(2, 0)
    transpose_store(1, 1, False)

    def body(m, carry):
        k = 2 * m
        fire_gathers(k + 1, 1)
        transpose_store(k, 0, True)
        fire_gathers(k + 2, 0)
        transpose_store(k + 1, 1, True)
        return carry

    lax.fori_loop(1, _NCHUNK // 2 - 1, body, 0)
    fire_gathers(_NCHUNK - 1, 1)
    transpose_store(_NCHUNK - 2, 0, True)
    transpose_store(_NCHUNK - 1, 1, True)
    drain_tiles(0)
    drain_tiles(1)


def kernel(token_ids, weight):
    b, p = token_ids.shape
    idx = token_ids.astype(jnp.int32).T.reshape(_NUM_ROWS // _L, _L)
    out = _emb_gather(weight, idx)
    out = out.reshape(p, _NBR, b // _L, 8, _L)
    out = out.transpose(0, 1, 3, 2, 4).reshape(p, _DIM, b).transpose(2, 0, 1)
    return out


# speedup vs baseline: 1.0450x; 1.0450x over previous
import functools

import jax
import jax.numpy as jnp
from jax import lax
from jax.experimental import pallas as pl
from jax.experimental.pallas import tpu as pltpu
from jax.experimental.pallas import tpu_sc as plsc

_NUM_ROWS = 16384 * 50
_DIM = 32
_NW = 32
_PER_W = _NUM_ROWS // _NW
_L = 128
_C = 512
_G = _C // _L
_NCHUNK = _PER_W // _C
_GROWS = _PER_W // _L
_NBR = _DIM // 8

_mesh = plsc.VectorSubcoreMesh(core_axis_name="c", subcore_axis_name="s")


@functools.partial(
    pl.kernel,
    mesh=_mesh,
    out_type=jax.ShapeDtypeStruct((_NUM_ROWS * _DIM,), jnp.float32),
    scratch_types=[
        pltpu.VMEM((_GROWS, _L), jnp.int32),
        pltpu.VMEM((2, _C, _DIM), jnp.float32),
        pltpu.VMEM((2, _G * _DIM * _L), jnp.float32),
        pltpu.SemaphoreType.DMA,
        pltpu.SemaphoreType.DMA,
        pltpu.SemaphoreType.DMA,
        pltpu.SemaphoreType.DMA,
    ],
    compiler_params=pltpu.CompilerParams(
        use_tc_tiling_on_sc=False, needs_layout_passes=False),
)
def _emb_gather(w_hbm, idx_hbm, out_hbm, idx_v, rows_v, tiles_v,
                gsem0, gsem1, ssem0, ssem1):
    gsems = (gsem0, gsem1)
    ssems = (ssem0, ssem1)
    wid = lax.axis_index("s") * 2 + lax.axis_index("c")
    gbase = pl.multiple_of(wid * _GROWS, 8)
    pltpu.sync_copy(idx_hbm.at[pl.ds(gbase, _GROWS)], idx_v)
    lanes = lax.iota(jnp.int32, 16)

    def fire_gathers(k, s):
        for j in range(_G):
            pltpu.async_copy(
                w_hbm.at[idx_v.at[k * _G + j]],
                rows_v.at[s].at[pl.ds(j * _L, _L)],
                gsems[s])

    def drain_rows(s):
        pltpu.make_async_copy(
            w_hbm.at[pl.ds(0, _C)], rows_v.at[s], gsems[s]).wait()

    def drain_tiles(s):
        pltpu.make_async_copy(
            out_hbm.at[pl.ds(0, _G * _DIM * _L)], tiles_v.at[s],
            ssems[s]).wait()

    def transpose_store(k, q, drain_stores):
        drain_rows(q)
        if drain_stores:
            drain_tiles(q)
        tiles = tiles_v.at[q]
        d128 = lanes * _L

        @plsc.parallel_loop(0, _C, 1, unroll=32)
        def tok_body(t):
            g = t // _L
            va = rows_v[q, t, pl.ds(0, 16)]
            vb = rows_v[q, t, pl.ds(16, 16)]
            base = d128 + (t + g * (_DIM * _L - _L))
            plsc.store_scatter(tiles, [base], va)
            plsc.store_scatter(tiles, [base + 16 * _L], vb)

        for g in range(_G):
            gg = wid * (_NCHUNK * _G) + k * _G + g
            p = gg // (16384 // _L)
            tc = lax.rem(gg, 16384 // _L)
            for br in range(_NBR):
                off = pl.multiple_of(
                    (p * _NBR * (16384 // _L) + br * (16384 // _L) + tc)
                    * (8 * _L), 8)
                pltpu.async_copy(
                    tiles.at[pl.ds(g * (_DIM * _L) + br * (8 * _L), 8 * _L)],
                    out_hbm.at[pl.ds(off, 8 * _L)],
                    ssems[q])

    fire_gathers(0, 0)
    fire_gathers(1, 1)
    transpose_store(0, 0, False)
    fire_gathers(2, 0)
    transpose_store(1, 1, False)

    def body(m, carry):
        k = 2 * m
        fire_gathers(k + 1, 1)
        transpose_store(k, 0, True)
        fire_gathers(k + 2, 0)
        transpose_store(k + 1, 1, True)
        return carry

    lax.fori_loop(1, _NCHUNK // 2 - 1, body, 0)
    fire_gathers(_NCHUNK - 1, 1)
    transpose_store(_NCHUNK - 2, 0, True)
    transpose_store(_NCHUNK - 1, 1, True)
    drain_tiles(0)
    drain_tiles(1)


def kernel(token_ids, weight):
    b, p = token_ids.shape
    idx = token_ids.astype(jnp.int32).T.reshape(_NUM_ROWS // _L, _L)
    out = _emb_gather(weight, idx)
    out = out.reshape(p, _NBR, b // _L, 8, _L)
    out = out.transpose(0, 1, 3, 2, 4).reshape(p, _DIM, b).transpose(2, 0, 1)
    return out


# speedup vs baseline: 1.0728x; 1.0266x over previous
import functools

import jax
import jax.numpy as jnp
from jax import lax
from jax.experimental import pallas as pl
from jax.experimental.pallas import tpu as pltpu
from jax.experimental.pallas import tpu_sc as plsc

_NUM_ROWS = 16384 * 50
_DIM = 32
_NW = 32
_PER_W = _NUM_ROWS // _NW
_L = 128
_C = 512
_G = _C // _L
_NCHUNK = _PER_W // _C
_GROWS = _PER_W // _L
_NBR = _DIM // 8

_mesh = plsc.VectorSubcoreMesh(core_axis_name="c", subcore_axis_name="s")


@functools.partial(
    pl.kernel,
    mesh=_mesh,
    out_type=jax.ShapeDtypeStruct((_NUM_ROWS * _DIM,), jnp.float32),
    scratch_types=[
        pltpu.VMEM((_GROWS, _L), jnp.int32),
        pltpu.VMEM((2, _C, _DIM), jnp.float32),
        pltpu.VMEM((2, _G * (_DIM // 2) * _L), jnp.float32),
        pltpu.VMEM((2, _G * (_DIM // 2) * _L), jnp.float32),
        pltpu.SemaphoreType.DMA,
        pltpu.SemaphoreType.DMA,
        pltpu.SemaphoreType.DMA,
        pltpu.SemaphoreType.DMA,
    ],
    compiler_params=pltpu.CompilerParams(
        use_tc_tiling_on_sc=False, needs_layout_passes=False),
)
def _emb_gather(w_hbm, idx_hbm, out_hbm, idx_v, rows_v, tiles_a, tiles_b,
                gsem0, gsem1, ssem0, ssem1):
    gsems = (gsem0, gsem1)
    ssems = (ssem0, ssem1)
    wid = lax.axis_index("s") * 2 + lax.axis_index("c")
    gbase = pl.multiple_of(wid * _GROWS, 8)
    pltpu.sync_copy(idx_hbm.at[pl.ds(gbase, _GROWS)], idx_v)
    lanes = lax.iota(jnp.int32, 16)

    def fire_gathers(k, s):
        for j in range(_G):
            pltpu.async_copy(
                w_hbm.at[idx_v.at[k * _G + j]],
                rows_v.at[s].at[pl.ds(j * _L, _L)],
                gsems[s])

    def drain_rows(s):
        pltpu.make_async_copy(
            w_hbm.at[pl.ds(0, _C)], rows_v.at[s], gsems[s]).wait()

    def drain_tiles(s):
        pltpu.make_async_copy(
            w_hbm.at[pl.ds(0, _C)], rows_v.at[s], ssems[s]).wait()

    def transpose_store(k, q, drain_stores):
        drain_rows(q)
        if drain_stores:
            drain_tiles(q)
        ta = tiles_a.at[q]
        tb = tiles_b.at[q]
        half = (_DIM // 2) * _L
        d128 = lanes * _L

        @plsc.parallel_loop(0, _C, 1, unroll=16)
        def tok_body(t):
            g = t // _L
            va = rows_v[q, t, pl.ds(0, 16)]
            vb = rows_v[q, t, pl.ds(16, 16)]
            base = d128 + (t + g * (half - _L))
            plsc.store_scatter(ta, [base], va)
            plsc.store_scatter(tb, [base], vb)

        for g in range(_G):
            gg = wid * (_NCHUNK * _G) + k * _G + g
            p = gg // (16384 // _L)
            tc = lax.rem(gg, 16384 // _L)
            for br in range(_NBR):
                src = (ta if br < _NBR // 2 else tb).at[
                    pl.ds(g * half + (br % (_NBR // 2)) * (8 * _L), 8 * _L)]
                off = pl.multiple_of(
                    (p * _NBR * (16384 // _L) + br * (16384 // _L) + tc)
                    * (8 * _L), 8)
                pltpu.async_copy(src, out_hbm.at[pl.ds(off, 8 * _L)], ssems[q])

    fire_gathers(0, 0)
    fire_gathers(1, 1)
    transpose_store(0, 0, False)
    fire_gathers(2, 0)
    transpose_store(1, 1, False)

    def body(m, carry):
        k = 2 * m
        fire_gathers(k + 1, 1)
        transpose_store(k, 0, True)
        fire_gathers(k + 2, 0)
        transpose_store(k + 1, 1, True)
        return carry

    lax.fori_loop(1, _NCHUNK // 2 - 1, body, 0)
    fire_gathers(_NCHUNK - 1, 1)
    transpose_store(_NCHUNK - 2, 0, True)
    transpose_store(_NCHUNK - 1, 1, True)
    drain_tiles(0)
    drain_tiles(1)


def kernel(token_ids, weight):
    b, p = token_ids.shape
    idx = token_ids.astype(jnp.int32).T.reshape(_NUM_ROWS // _L, _L)
    out = _emb_gather(weight, idx)
    out = out.reshape(p, _NBR, b // _L, 8, _L)
    out = out.transpose(0, 1, 3, 2, 4).reshape(p, _DIM, b).transpose(2, 0, 1)
    return out


# speedup vs baseline: 1.0853x; 1.0117x over previous
import functools

import jax
import jax.numpy as jnp
from jax import lax
from jax.experimental import pallas as pl
from jax.experimental.pallas import tpu as pltpu
from jax.experimental.pallas import tpu_sc as plsc

_NUM_ROWS = 16384 * 50
_DIM = 32
_NW = 32
_PER_W = _NUM_ROWS // _NW
_L = 128
_C = 512
_G = _C // _L
_NCHUNK = _PER_W // _C
_GROWS = _PER_W // _L
_NBR = _DIM // 8

_mesh = plsc.VectorSubcoreMesh(core_axis_name="c", subcore_axis_name="s")


@functools.partial(
    pl.kernel,
    mesh=_mesh,
    out_type=jax.ShapeDtypeStruct((_NUM_ROWS * _DIM,), jnp.float32),
    scratch_types=[
        pltpu.VMEM((_GROWS, _L), jnp.int32),
        pltpu.VMEM((2, _C, _DIM), jnp.float32),
        pltpu.VMEM((2, _G * _DIM * _L), jnp.float32),
        pltpu.SemaphoreType.DMA,
        pltpu.SemaphoreType.DMA,
        pltpu.SemaphoreType.DMA,
        pltpu.SemaphoreType.DMA,
    ],
    compiler_params=pltpu.CompilerParams(
        use_tc_tiling_on_sc=False, needs_layout_passes=False),
)
def _emb_gather(w_hbm, idx_hbm, out_hbm, idx_v, rows_v, tiles_v,
                gsem0, gsem1, ssem0, ssem1):
    gsems = (gsem0, gsem1)
    ssems = (ssem0, ssem1)
    wid = lax.axis_index("s") * 2 + lax.axis_index("c")
    gbase = pl.multiple_of(wid * _GROWS, 8)
    pltpu.sync_copy(idx_hbm.at[pl.ds(gbase, _GROWS)], idx_v)
    lanes = lax.iota(jnp.int32, 16)

    def fire_gathers(k, s):
        for j in range(_G):
            pltpu.async_copy(
                w_hbm.at[idx_v.at[k * _G + j]],
                rows_v.at[s].at[pl.ds(j * _L, _L)],
                gsems[s])

    def drain_rows(s):
        pltpu.make_async_copy(
            w_hbm.at[pl.ds(0, _C)], rows_v.at[s], gsems[s]).wait()

    def drain_tiles(s):
        pltpu.make_async_copy(
            out_hbm.at[pl.ds(0, _G * _DIM * _L)], tiles_v.at[s],
            ssems[s]).wait()

    def transpose_store(k, q, drain_stores):
        drain_rows(q)
        if drain_stores:
            drain_tiles(q)
        tiles = tiles_v.at[q]
        d128 = lanes * _L

        @plsc.parallel_loop(0, _C, 1, unroll=16)
        def tok_body(t):
            g = t // _L
            va = rows_v[q, t, pl.ds(0, 16)]
            vb = rows_v[q, t, pl.ds(16, 16)]
            base = d128 + (t + g * (_DIM * _L - _L))
            plsc.store_scatter(tiles, [base], va)
            plsc.store_scatter(tiles, [base + 16 * _L], vb)

        for g in range(_G):
            gg = wid * (_NCHUNK * _G) + k * _G + g
            p = gg // (16384 // _L)
            tc = lax.rem(gg, 16384 // _L)
            for br in range(_NBR):
                off = pl.multiple_of(
                    (p * _NBR * (16384 // _L) + br * (16384 // _L) + tc)
                    * (8 * _L), 8)
                pltpu.async_copy(
                    tiles.at[pl.ds(g * (_DIM * _L) + br * (8 * _L), 8 * _L)],
                    out_hbm.at[pl.ds(off, 8 * _L)],
                    ssems[q])

    fire_gathers(0, 0)
    fire_gathers(1, 1)
    transpose_store(0, 0, False)
    fire_gathers(2, 0)
    transpose_store(1, 1, False)

    def body(m, carry):
        k = 2 * m
        fire_gathers(k + 1, 1)
        transpose_store(k, 0, True)
        fire_gathers(k + 2, 0)
        transpose_store(k + 1, 1, True)
        return carry

    lax.fori_loop(1, _NCHUNK // 2 - 1, body, 0)
    fire_gathers(_NCHUNK - 1, 1)
    transpose_store(_NCHUNK - 2, 0, True)
    transpose_store(_NCHUNK - 1, 1, True)
    drain_tiles(0)
    drain_tiles(1)


def kernel(token_ids, weight):
    b, p = token_ids.shape
    idx = token_ids.astype(jnp.int32).T.reshape(_NUM_ROWS // _L, _L)
    out = _emb_gather(weight, idx)
    out = out.reshape(p, _NBR, b // _L, 8, _L)
    out = out.transpose(0, 1, 3, 2, 4).reshape(p, _DIM, b).transpose(2, 0, 1)
    return out
